# baseline (speedup 1.0000x reference)
import functools

import jax
import jax.numpy as jnp
from jax.experimental import pallas as pl

_B, _T, _C, _NH = 1, 2048, 768, 32
_HD = _C // _NH
_BQ = 128
_SCALE = 1.0 / (_HD ** 0.5)


def _qkv_kernel(x_ref, w_ref, o_ref):
    o_ref[...] = jax.lax.dot_general(
        x_ref[...], w_ref[...],
        dimension_numbers=(((1,), (1,)), ((), ())),
        preferred_element_type=jnp.float32,
    ).astype(jnp.bfloat16)


def _attn_kernel(q_ref, k_ref, v_ref, wp_ref, o_ref, *, base, ext):
    iq = base + pl.program_id(0)
    row = iq * _BQ + jax.lax.broadcasted_iota(jnp.int32, (_BQ, ext), 0)
    col = jax.lax.broadcasted_iota(jnp.int32, (_BQ, ext), 1)
    keep = col <= row
    outs = []
    for h in range(_NH):
        q = q_ref[:, h * _HD:(h + 1) * _HD]
        k = k_ref[:, h * _HD:(h + 1) * _HD]
        v = v_ref[:, h * _HD:(h + 1) * _HD]
        s = jax.lax.dot_general(
            q, k, dimension_numbers=(((1,), (1,)), ((), ())),
            preferred_element_type=jnp.float32,
        ) * _SCALE
        s = jnp.where(keep, s, -jnp.inf)
        m = jnp.max(s, axis=1, keepdims=True)
        p = jnp.exp(s - m)
        l = jnp.sum(p, axis=1, keepdims=True)
        o = jnp.dot(p.astype(jnp.bfloat16), v,
                    preferred_element_type=jnp.float32) / l
        outs.append(o)
    y = jnp.concatenate(outs, axis=1)
    o_ref[...] = jax.lax.dot_general(
        y, wp_ref[...], dimension_numbers=(((1,), (1,)), ((), ())),
        preferred_element_type=jnp.float32,
    )


def _attention(qkv, W_proj):
    outs = []
    nb = 512 // _BQ
    for g in range(_T // 512):
        ext = (g + 1) * 512
        out_g = pl.pallas_call(
            functools.partial(_attn_kernel, base=nb * g, ext=ext),
            grid=(nb,),
            in_specs=[
                pl.BlockSpec((_BQ, _C), lambda i, g=g, nb=nb: (nb * g + i, 0)),
                pl.BlockSpec((ext, _C), lambda i: (0, 1)),
                pl.BlockSpec((ext, _C), lambda i: (0, 2)),
                pl.BlockSpec((_C, _C), lambda i: (0, 0)),
            ],
            out_specs=pl.BlockSpec((_BQ, _C), lambda i: (i, 0)),
            out_shape=jax.ShapeDtypeStruct((512, _C), jnp.float32),
        )(qkv, qkv, qkv, W_proj)
        outs.append(out_g)
    return jnp.concatenate(outs, axis=0)


def kernel(x, W_attn, W_proj):
    b, t, c = x.shape
    x2 = x.reshape(t, c)
    qkv = pl.pallas_call(
        _qkv_kernel,
        out_shape=jax.ShapeDtypeStruct((t, 3 * c), jnp.bfloat16),
    )(x2, W_attn)
    out = _attention(qkv, W_proj)
    return out.reshape(b, t, c)


# speedup vs baseline: 1.2165x; 1.2165x over previous
import functools

import jax
import jax.numpy as jnp
from jax.experimental import pallas as pl

_B, _T, _C, _NH = 1, 2048, 768, 32
_HD = _C // _NH
_BQ = 256
_SCALE = 1.0 / (_HD ** 0.5)


def _qkv_kernel(x_ref, w_ref, o_ref):
    qkv = jax.lax.dot_general(
        x_ref[...], w_ref[...],
        dimension_numbers=(((1,), (1,)), ((), ())),
        preferred_element_type=jnp.float32,
    )
    scale = jnp.concatenate([
        jnp.full((1, _C), _SCALE, jnp.float32),
        jnp.ones((1, 2 * _C), jnp.float32),
    ], axis=1)
    o_ref[...] = (qkv * scale).astype(jnp.bfloat16)


def _proj_kernel(x_ref, w_ref, o_ref):
    o_ref[...] = jax.lax.dot_general(
        x_ref[...], w_ref[...],
        dimension_numbers=(((1,), (1,)), ((), ())),
        preferred_element_type=jnp.float32,
    )


def _attn_kernel(q_ref, k_ref, v_ref, o_ref, *, base, ext):
    iq = base + pl.program_id(0)
    row = iq * _BQ + jax.lax.broadcasted_iota(jnp.int32, (_BQ, ext), 0)
    col = jax.lax.broadcasted_iota(jnp.int32, (_BQ, ext), 1)
    keep = col <= row
    for h in range(_NH):
        q = q_ref[:, h * _HD:(h + 1) * _HD]
        k = k_ref[:, h * _HD:(h + 1) * _HD]
        v = v_ref[:, h * _HD:(h + 1) * _HD]
        s = jax.lax.dot_general(
            q, k, dimension_numbers=(((1,), (1,)), ((), ())),
            preferred_element_type=jnp.float32,
        )
        s = jnp.where(keep, s, -jnp.inf)
        m = jnp.max(s, axis=1, keepdims=True)
        p = jnp.exp(s - m)
        l = jnp.sum(p, axis=1, keepdims=True)
        o = jnp.dot(p.astype(jnp.bfloat16), v,
                    preferred_element_type=jnp.float32) / l
        o_ref[:, h * _HD:(h + 1) * _HD] = o


def _attention(qkv):
    outs = []
    for g in range(_T // (2 * _BQ)):
        ext = (2 * g + 2) * _BQ
        out_g = pl.pallas_call(
            functools.partial(_attn_kernel, base=2 * g, ext=ext),
            grid=(2,),
            in_specs=[
                pl.BlockSpec((_BQ, _C), lambda i, g=g: (2 * g + i, 0)),
                pl.BlockSpec((ext, _C), lambda i: (0, 1)),
                pl.BlockSpec((ext, _C), lambda i: (0, 2)),
            ],
            out_specs=pl.BlockSpec((_BQ, _C), lambda i: (i, 0)),
            out_shape=jax.ShapeDtypeStruct((2 * _BQ, _C), jnp.float32),
        )(qkv, qkv, qkv)
        outs.append(out_g)
    return jnp.concatenate(outs, axis=0)


def kernel(x, W_attn, W_proj):
    b, t, c = x.shape
    x2 = x.reshape(t, c)
    qkv = pl.pallas_call(
        _qkv_kernel,
        out_shape=jax.ShapeDtypeStruct((t, 3 * c), jnp.bfloat16),
    )(x2, W_attn)
    y = _attention(qkv)
    out = pl.pallas_call(
        _proj_kernel,
        out_shape=jax.ShapeDtypeStruct((t, c), jnp.float32),
    )(y, W_proj)
    return out.reshape(b, t, c)


# speedup vs baseline: 1.3011x; 1.0695x over previous
import functools

import jax
import jax.numpy as jnp
from jax.experimental import pallas as pl

_B, _T, _C, _NH = 1, 2048, 768, 32
_HD = _C // _NH
_BQ = 256
_LOG2E = 1.4426950408889634
_SCALE = _LOG2E / (_HD ** 0.5)


def _qkv_kernel(x_ref, w_ref, o_ref):
    qkv = jax.lax.dot_general(
        x_ref[...], w_ref[...],
        dimension_numbers=(((1,), (1,)), ((), ())),
        preferred_element_type=jnp.float32,
    )
    scale = jnp.concatenate([
        jnp.full((1, _C), _SCALE, jnp.float32),
        jnp.ones((1, 2 * _C), jnp.float32),
    ], axis=1)
    o_ref[...] = (qkv * scale).astype(jnp.bfloat16)


def _proj_kernel(x_ref, w_ref, o_ref):
    o_ref[...] = jax.lax.dot_general(
        x_ref[...], w_ref[...],
        dimension_numbers=(((1,), (1,)), ((), ())),
        preferred_element_type=jnp.float32,
    )


def _attn_kernel(q_ref, k_ref, v_ref, o_ref, *, ext):
    rloc = jax.lax.broadcasted_iota(jnp.int32, (_BQ, _BQ), 0)
    cloc = jax.lax.broadcasted_iota(jnp.int32, (_BQ, _BQ), 1)
    keep = cloc <= rloc
    main = ext - _BQ
    ones_col = jnp.ones((ext, 1), jnp.bfloat16)
    for h in range(_NH):
        q = q_ref[:, h * _HD:(h + 1) * _HD]
        k = k_ref[:, h * _HD:(h + 1) * _HD]
        v = v_ref[:, h * _HD:(h + 1) * _HD]
        va = jnp.concatenate([v, ones_col], axis=1)
        sb = jax.lax.dot_general(
            q, k[main:, :], dimension_numbers=(((1,), (1,)), ((), ())),
            preferred_element_type=jnp.float32,
        )
        sb = jnp.where(keep, sb, -jnp.inf)
        mb = jnp.max(sb, axis=1, keepdims=True)
        if main:
            sm = jax.lax.dot_general(
                q, k[:main, :], dimension_numbers=(((1,), (1,)), ((), ())),
                preferred_element_type=jnp.float32,
            )
            m = jnp.maximum(jnp.max(sm, axis=1, keepdims=True), mb)
            pm = jnp.exp2(sm - m).astype(jnp.bfloat16)
            pb = jnp.exp2(sb - m).astype(jnp.bfloat16)
            ol = jnp.dot(pm, va[:main, :],
                         preferred_element_type=jnp.float32)
            ol = ol + jnp.dot(pb, va[main:, :],
                              preferred_element_type=jnp.float32)
        else:
            pb = jnp.exp2(sb - mb).astype(jnp.bfloat16)
            ol = jnp.dot(pb, va, preferred_element_type=jnp.float32)
        o = ol[:, :_HD] * (1.0 / ol[:, _HD:])
        o_ref[:, h * _HD:(h + 1) * _HD] = o


def _attention(qkv):
    outs = []
    for i in range(_T // _BQ):
        ext = (i + 1) * _BQ
        out_i = pl.pallas_call(
            functools.partial(_attn_kernel, ext=ext),
            grid=(1,),
            in_specs=[
                pl.BlockSpec((_BQ, _C), lambda j, i=i: (i, 0)),
                pl.BlockSpec((ext, _C), lambda j: (0, 1)),
                pl.BlockSpec((ext, _C), lambda j: (0, 2)),
            ],
            out_specs=pl.BlockSpec((_BQ, _C), lambda j: (0, 0)),
            out_shape=jax.ShapeDtypeStruct((_BQ, _C), jnp.float32),
        )(qkv, qkv, qkv)
        outs.append(out_i)
    return jnp.concatenate(outs, axis=0)


def kernel(x, W_attn, W_proj):
    b, t, c = x.shape
    x2 = x.reshape(t, c)
    qkv = pl.pallas_call(
        _qkv_kernel,
        out_shape=jax.ShapeDtypeStruct((t, 3 * c), jnp.bfloat16),
    )(x2, W_attn)
    y = _attention(qkv)
    out = pl.pallas_call(
        _proj_kernel,
        out_shape=jax.ShapeDtypeStruct((t, c), jnp.float32),
    )(y, W_proj)
    return out.reshape(b, t, c)
